# baseline (device time: 743503 ns/iter reference)
import jax
import jax.numpy as jnp
from jax import lax
from jax.experimental import pallas as pl
from jax.experimental.pallas import tpu as pltpu

N_DEV = 16
COMM_DTYPE = jnp.bfloat16


def kernel(x, w_mat, scale_x, scale_w):
    m, _k_per = x.shape
    _, n = w_mat.shape
    m_blk = m // N_DEV
    scale = (scale_x * scale_w).reshape(1, 1)

    def body(x_ref, w_ref, scale_ref, out_ref,
             send_ref, comm_ref, send_sems, recv_sems, credit_sem):
        my = lax.axis_index("i")
        left = lax.rem(my + N_DEV - 1, N_DEV)
        right = lax.rem(my + 1, N_DEV)

        barrier = pltpu.get_barrier_semaphore()
        for nbr in (left, right):
            pl.semaphore_signal(
                barrier, inc=1,
                device_id=(nbr,), device_id_type=pl.DeviceIdType.MESH,
            )
        pl.semaphore_wait(barrier, 2)

        def partial_for(s):
            c = lax.rem(my + 2 * N_DEV - s - 1, N_DEV)
            xc = x_ref[pl.ds(c * m_blk, m_blk), :]
            return jnp.dot(xc, w_ref[:, :], preferred_element_type=jnp.int32)

        def desc(slot):
            return pltpu.make_async_remote_copy(
                src_ref=send_ref.at[slot],
                dst_ref=comm_ref.at[slot],
                send_sem=send_sems.at[slot],
                recv_sem=recv_sems.at[slot],
                device_id=(right,),
                device_id_type=pl.DeviceIdType.MESH,
            )

        send_ref[0] = partial_for(0).astype(COMM_DTYPE)
        desc(0).start()

        for s in range(1, N_DEV):
            partial = partial_for(s).astype(jnp.float32)
            in_slot = (s - 1) % 2
            desc(in_slot).wait_recv()
            acc = comm_ref[in_slot].astype(jnp.float32) + partial
            if s < N_DEV - 1:
                slot = s % 2
                if s >= 2:
                    desc(slot).wait_send()
                    pl.semaphore_wait(credit_sem, 1)
                send_ref[slot] = acc.astype(COMM_DTYPE)
                if s <= N_DEV - 3:
                    pl.semaphore_signal(
                        credit_sem, inc=1,
                        device_id=(left,), device_id_type=pl.DeviceIdType.MESH,
                    )
                desc(slot).start()
            else:
                y = acc * scale_ref[0, 0]
                out_ref[:, :] = y * jax.nn.sigmoid(jnp.clip(y, -60.0, 60.0))

        desc(0).wait_send()
        desc(1).wait_send()

    return pl.pallas_call(
        body,
        out_shape=jax.ShapeDtypeStruct((m_blk, n), jnp.float32),
        in_specs=[
            pl.BlockSpec(memory_space=pltpu.VMEM),
            pl.BlockSpec(memory_space=pltpu.VMEM),
            pl.BlockSpec(memory_space=pltpu.SMEM),
        ],
        out_specs=pl.BlockSpec(memory_space=pltpu.VMEM),
        scratch_shapes=[
            pltpu.VMEM((2, m_blk, n), COMM_DTYPE),
            pltpu.VMEM((2, m_blk, n), COMM_DTYPE),
            pltpu.SemaphoreType.DMA((2,)),
            pltpu.SemaphoreType.DMA((2,)),
            pltpu.SemaphoreType.REGULAR,
        ],
        compiler_params=pltpu.CompilerParams(collective_id=0),
    )(x, w_mat, scale)


# device time: 423849 ns/iter; 1.7542x vs baseline; 1.7542x over previous
import jax
import jax.numpy as jnp
from jax import lax
from jax.experimental import pallas as pl
from jax.experimental.pallas import tpu as pltpu

N_DEV = 16
COMM_DTYPE = jnp.bfloat16


def kernel(x, w_mat, scale_x, scale_w):
    m, _k_per = x.shape
    _, n = w_mat.shape
    m_blk = m // N_DEV
    n_half = n // 2
    scale = (scale_x * scale_w).reshape(1, 1)

    def body(x_ref, w_ref, scale_ref, out_ref,
             send_cw, comm_cw, send_ccw, comm_ccw,
             ssem_cw, rsem_cw, ssem_ccw, rsem_ccw,
             credit_cw, credit_ccw):
        my = lax.axis_index("i")
        left = lax.rem(my + N_DEV - 1, N_DEV)
        right = lax.rem(my + 1, N_DEV)

        barrier = pltpu.get_barrier_semaphore()
        for nbr in (left, right):
            pl.semaphore_signal(
                barrier, inc=1,
                device_id=(nbr,), device_id_type=pl.DeviceIdType.MESH,
            )
        pl.semaphore_wait(barrier, 2)

        def partial_cw(s):
            c = lax.rem(my + 2 * N_DEV - s - 1, N_DEV)
            xc = x_ref[pl.ds(c * m_blk, m_blk), :]
            return jnp.dot(xc, w_ref[:, :n_half],
                           preferred_element_type=jnp.int32)

        def partial_ccw(s):
            c = lax.rem(my + s + 1, N_DEV)
            xc = x_ref[pl.ds(c * m_blk, m_blk), :]
            return jnp.dot(xc, w_ref[:, n_half:],
                           preferred_element_type=jnp.int32)

        def desc_cw(slot):
            return pltpu.make_async_remote_copy(
                src_ref=send_cw.at[slot], dst_ref=comm_cw.at[slot],
                send_sem=ssem_cw.at[slot], recv_sem=rsem_cw.at[slot],
                device_id=(right,), device_id_type=pl.DeviceIdType.MESH,
            )

        def desc_ccw(slot):
            return pltpu.make_async_remote_copy(
                src_ref=send_ccw.at[slot], dst_ref=comm_ccw.at[slot],
                send_sem=ssem_ccw.at[slot], recv_sem=rsem_ccw.at[slot],
                device_id=(left,), device_id_type=pl.DeviceIdType.MESH,
            )

        send_cw[0] = partial_cw(0).astype(COMM_DTYPE)
        desc_cw(0).start()
        send_ccw[0] = partial_ccw(0).astype(COMM_DTYPE)
        desc_ccw(0).start()

        for s in range(1, N_DEV):
            p_cw = partial_cw(s).astype(jnp.float32)
            p_ccw = partial_ccw(s).astype(jnp.float32)
            in_slot = (s - 1) % 2
            desc_cw(in_slot).wait_recv()
            desc_ccw(in_slot).wait_recv()
            acc_cw = comm_cw[in_slot].astype(jnp.float32) + p_cw
            acc_ccw = comm_ccw[in_slot].astype(jnp.float32) + p_ccw
            if s < N_DEV - 1:
                slot = s % 2
                if s >= 2:
                    desc_cw(slot).wait_send()
                    desc_ccw(slot).wait_send()
                    pl.semaphore_wait(credit_cw, 1)
                    pl.semaphore_wait(credit_ccw, 1)
                send_cw[slot] = acc_cw.astype(COMM_DTYPE)
                send_ccw[slot] = acc_ccw.astype(COMM_DTYPE)
                if s <= N_DEV - 3:
                    pl.semaphore_signal(
                        credit_cw, inc=1,
                        device_id=(left,), device_id_type=pl.DeviceIdType.MESH,
                    )
                    pl.semaphore_signal(
                        credit_ccw, inc=1,
                        device_id=(right,), device_id_type=pl.DeviceIdType.MESH,
                    )
                desc_cw(slot).start()
                desc_ccw(slot).start()
            else:
                sc = scale_ref[0, 0]
                y0 = acc_cw * sc
                out_ref[:, :n_half] = y0 * jax.nn.sigmoid(
                    jnp.clip(y0, -60.0, 60.0))
                y1 = acc_ccw * sc
                out_ref[:, n_half:] = y1 * jax.nn.sigmoid(
                    jnp.clip(y1, -60.0, 60.0))

        desc_cw(0).wait_send()
        desc_cw(1).wait_send()
        desc_ccw(0).wait_send()
        desc_ccw(1).wait_send()

    return pl.pallas_call(
        body,
        out_shape=jax.ShapeDtypeStruct((m_blk, n), jnp.float32),
        in_specs=[
            pl.BlockSpec(memory_space=pltpu.VMEM),
            pl.BlockSpec(memory_space=pltpu.VMEM),
            pl.BlockSpec(memory_space=pltpu.SMEM),
        ],
        out_specs=pl.BlockSpec(memory_space=pltpu.VMEM),
        scratch_shapes=[
            pltpu.VMEM((2, m_blk, n_half), COMM_DTYPE),
            pltpu.VMEM((2, m_blk, n_half), COMM_DTYPE),
            pltpu.VMEM((2, m_blk, n_half), COMM_DTYPE),
            pltpu.VMEM((2, m_blk, n_half), COMM_DTYPE),
            pltpu.SemaphoreType.DMA((2,)),
            pltpu.SemaphoreType.DMA((2,)),
            pltpu.SemaphoreType.DMA((2,)),
            pltpu.SemaphoreType.DMA((2,)),
            pltpu.SemaphoreType.REGULAR,
            pltpu.SemaphoreType.REGULAR,
        ],
        compiler_params=pltpu.CompilerParams(collective_id=0),
    )(x, w_mat, scale)


# device time: 353175 ns/iter; 2.1052x vs baseline; 1.2001x over previous
import jax
import jax.numpy as jnp
from jax import lax
from jax.experimental import pallas as pl
from jax.experimental.pallas import tpu as pltpu

N_DEV = 16
N_STREAM = 4
COMM_DTYPE = jnp.bfloat16


def kernel(x, w_mat, scale_x, scale_w):
    m, _k_per = x.shape
    _, n = w_mat.shape
    m_blk = m // N_DEV
    nq = n // N_STREAM
    scale = (scale_x * scale_w).reshape(1, 1)

    def body(x_ref, w_ref, scale_ref, out_ref, *scratch):
        sends = scratch[0:4]
        comms = scratch[4:8]
        ssems = scratch[8:12]
        rsems = scratch[12:16]
        credits = scratch[16:20]

        my = lax.axis_index("i")
        left = lax.rem(my + N_DEV - 1, N_DEV)
        right = lax.rem(my + 1, N_DEV)
        dst = (right, right, left, left)
        ups = (left, left, right, right)

        barrier = pltpu.get_barrier_semaphore()
        for nbr in (left, right):
            pl.semaphore_signal(
                barrier, inc=1,
                device_id=(nbr,), device_id_type=pl.DeviceIdType.MESH,
            )
        pl.semaphore_wait(barrier, 2)

        def partial(s, q):
            if q < 2:
                c = lax.rem(my + 2 * N_DEV - s - 1, N_DEV)
            else:
                c = lax.rem(my + s + 1, N_DEV)
            xc = x_ref[pl.ds(c * m_blk, m_blk), :]
            return jnp.dot(xc, w_ref[:, q * nq:(q + 1) * nq],
                           preferred_element_type=jnp.int32)

        def desc(q, slot):
            return pltpu.make_async_remote_copy(
                src_ref=sends[q].at[slot], dst_ref=comms[q].at[slot],
                send_sem=ssems[q].at[slot], recv_sem=rsems[q].at[slot],
                device_id=(dst[q],), device_id_type=pl.DeviceIdType.MESH,
            )

        for q in (0, 2, 1, 3):
            sends[q][0] = partial(0, q).astype(COMM_DTYPE)
            desc(q, 0).start()

        for s in range(1, N_DEV):
            parts = [partial(s, q).astype(jnp.float32) for q in range(4)]
            in_slot = (s - 1) % 2
            slot = s % 2
            for q in (0, 2, 1, 3):
                desc(q, in_slot).wait_recv()
                acc = comms[q][in_slot].astype(jnp.float32) + parts[q]
                if s < N_DEV - 1:
                    if s >= 2:
                        desc(q, slot).wait_send()
                        pl.semaphore_wait(credits[q], 1)
                    sends[q][slot] = acc.astype(COMM_DTYPE)
                    if s <= N_DEV - 3:
                        pl.semaphore_signal(
                            credits[q], inc=1,
                            device_id=(ups[q],),
                            device_id_type=pl.DeviceIdType.MESH,
                        )
                    desc(q, slot).start()
                else:
                    y = acc * scale_ref[0, 0]
                    out_ref[:, q * nq:(q + 1) * nq] = y * jax.nn.sigmoid(
                        jnp.clip(y, -60.0, 60.0))

        for q in range(4):
            desc(q, 0).wait_send()
            desc(q, 1).wait_send()

    return pl.pallas_call(
        body,
        out_shape=jax.ShapeDtypeStruct((m_blk, n), jnp.float32),
        in_specs=[
            pl.BlockSpec(memory_space=pltpu.VMEM),
            pl.BlockSpec(memory_space=pltpu.VMEM),
            pl.BlockSpec(memory_space=pltpu.SMEM),
        ],
        out_specs=pl.BlockSpec(memory_space=pltpu.VMEM),
        scratch_shapes=(
            [pltpu.VMEM((2, m_blk, nq), COMM_DTYPE) for _ in range(4)]
            + [pltpu.VMEM((2, m_blk, nq), COMM_DTYPE) for _ in range(4)]
            + [pltpu.SemaphoreType.DMA((2,)) for _ in range(4)]
            + [pltpu.SemaphoreType.DMA((2,)) for _ in range(4)]
            + [pltpu.SemaphoreType.REGULAR for _ in range(4)]
        ),
        compiler_params=pltpu.CompilerParams(collective_id=0),
    )(x, w_mat, scale)
